# baseline (speedup 1.0000x reference)
import jax
import jax.numpy as jnp
from jax.experimental import pallas as pl
from jax.experimental.pallas import tpu as pltpu


def _cdiv(a, b):
    return -(-a // b)


def _colsum_kernel(b_ref, xt_ref, w_ref, o_ref):
    acc = jnp.sum(xt_ref[...] * w_ref[...], axis=0)
    o_ref[...] = (acc + b_ref[0, 0]).astype(o_ref.dtype)


def kernel(x, weights, bias):
    N, D = x.shape
    w_col = jnp.asarray(weights, jnp.float32).reshape(D, 1)
    b_f32 = jnp.asarray(bias, jnp.float32).reshape(1, 1)
    xt = x.T
    itemsize = jnp.dtype(x.dtype).itemsize

    budget = 48 << 20
    tile_n = (budget // (2 * (D + 1) * itemsize)) // 1024 * 1024
    tile_n = max(1024, min(tile_n, _cdiv(N, 1024) * 1024))
    for cand in (131072, 65536, 32768, 16384, 8192):
        if cand <= tile_n and N % cand == 0:
            tile_n = cand
            break
    grid = _cdiv(N, tile_n)

    out = pl.pallas_call(
        _colsum_kernel,
        out_shape=jax.ShapeDtypeStruct((N,), x.dtype),
        grid=(grid,),
        in_specs=[
            pl.BlockSpec(memory_space=pltpu.SMEM),
            pl.BlockSpec((D, tile_n), lambda i: (0, i)),
            pl.BlockSpec((D, 1), lambda i: (0, 0)),
        ],
        out_specs=pl.BlockSpec((tile_n,), lambda i: (i,)),
        compiler_params=pltpu.CompilerParams(
            dimension_semantics=("parallel",)),
        cost_estimate=pl.CostEstimate(
            flops=2 * N * D, transcendentals=0,
            bytes_accessed=N * D * itemsize + N * itemsize),
    )(b_f32, xt, w_col)
    return out


# speedup vs baseline: 1.0161x; 1.0161x over previous
import jax
import jax.numpy as jnp
from jax.experimental import pallas as pl
from jax.experimental.pallas import tpu as pltpu


def _cdiv(a, b):
    return -(-a // b)


def _colsum_kernel(b_ref, xt_ref, w_ref, o_ref):
    w_col = w_ref[...].T
    acc = jnp.sum(xt_ref[...] * w_col, axis=0)
    o_ref[...] = (acc + b_ref[0, 0]).astype(o_ref.dtype)


def kernel(x, weights, bias):
    N, D = x.shape
    w_row = jnp.asarray(weights, jnp.float32).reshape(1, D)
    b_f32 = jnp.asarray(bias, jnp.float32).reshape(1, 1)
    xt = x.T
    itemsize = jnp.dtype(x.dtype).itemsize

    budget = 48 << 20
    tile_n = (budget // (2 * (D + 1) * itemsize)) // 1024 * 1024
    tile_n = max(1024, min(tile_n, _cdiv(N, 1024) * 1024))
    for cand in (131072, 65536, 32768, 16384, 8192):
        if cand <= tile_n and N % cand == 0:
            tile_n = cand
            break
    grid = _cdiv(N, tile_n)

    out = pl.pallas_call(
        _colsum_kernel,
        out_shape=jax.ShapeDtypeStruct((N,), x.dtype),
        grid=(grid,),
        in_specs=[
            pl.BlockSpec(memory_space=pltpu.SMEM),
            pl.BlockSpec((D, tile_n), lambda i: (0, i)),
            pl.BlockSpec((1, D), lambda i: (0, 0)),
        ],
        out_specs=pl.BlockSpec((tile_n,), lambda i: (i,)),
        compiler_params=pltpu.CompilerParams(
            dimension_semantics=("parallel",)),
        cost_estimate=pl.CostEstimate(
            flops=2 * N * D, transcendentals=0,
            bytes_accessed=N * D * itemsize + N * itemsize),
    )(b_f32, xt, w_row)
    return out
